# baseline (speedup 1.0000x reference)
import functools

import numpy as np
import jax
import jax.numpy as jnp
from jax import lax
from jax.experimental import pallas as pl


def _i32(v) -> int:
    return int(np.uint32(v).view(np.int32))


_K1 = 0
_K2 = 42
_KS2 = _i32(np.uint32(_K1) ^ np.uint32(_K2) ^ np.uint32(0x1BD11BDA))
_ROTS = ((13, 15, 26, 6), (17, 29, 16, 24))
_KS = (_i32(_K1), _i32(_K2), _KS2)
_TINY = float(np.finfo(np.float32).tiny)
_EXP_ONE = _i32(0x3F800000)
_BIG_I32 = np.iinfo(np.int32).max


def _rotl(x, r):
    return lax.shift_left(x, np.int32(r)) | lax.shift_right_logical(
        x, np.int32(32 - r))


def _threefry_bits(i):
    x0 = jnp.full(i.shape, _KS[0], jnp.int32)
    x1 = i + np.int32(_KS[1])
    for rnd in range(5):
        for r in _ROTS[rnd % 2]:
            x0 = x0 + x1
            x1 = _rotl(x1, r)
            x1 = x1 ^ x0
        x0 = x0 + np.int32(_KS[(rnd + 1) % 3])
        x1 = x1 + np.int32(_i32(np.uint32(_KS[(rnd + 2) % 3]) + np.uint32(rnd + 1)))
    return x0 ^ x1


def _sample_kernel(x_ref, idx_ref, val_ref, *, ncols, bc, sub):
    j = pl.program_id(0)
    base_col = j * np.int32(bc)
    bm_acc = None
    bi_acc = None
    for s in range(bc // sub):
        x = x_ref[:, s * sub:(s + 1) * sub]
        gcol = (lax.broadcasted_iota(jnp.int32, x.shape, 1)
                + (base_col + np.int32(s * sub)))
        row = lax.broadcasted_iota(jnp.int32, x.shape, 0)
        i = row * np.int32(ncols) + gcol

        bits = _threefry_bits(i)
        fb = lax.shift_right_logical(bits, np.int32(9)) | np.int32(_EXP_ONE)
        f = lax.bitcast_convert_type(fb, jnp.float32) - np.float32(1.0)
        u = jnp.maximum(f, np.float32(_TINY))
        g = -jnp.log(-jnp.log(u))

        v = jnp.where(gcol < np.int32(ncols), x + g, -jnp.inf)
        bm = jnp.max(v, axis=1, keepdims=True)
        cand = jnp.where(v == bm, gcol, np.int32(_BIG_I32))
        bi = jnp.min(cand, axis=1, keepdims=True)
        if bm_acc is None:
            bm_acc, bi_acc = bm, bi
        else:
            better = bm > bm_acc
            bm_acc = jnp.where(better, bm, bm_acc)
            bi_acc = jnp.where(better, bi, bi_acc)

    @pl.when(j == 0)
    def _():
        val_ref[...] = bm_acc
        idx_ref[...] = bi_acc

    @pl.when(j != 0)
    def _():
        better = bm_acc > val_ref[...]
        val_ref[...] = jnp.where(better, bm_acc, val_ref[...])
        idx_ref[...] = jnp.where(better, bi_acc, idx_ref[...])


@functools.partial(jax.jit, static_argnames=("block_cols", "sub_cols"))
def _sample(outputs, block_cols=49152, sub_cols=1024):
    rows, ncols = outputs.shape
    nblk = pl.cdiv(ncols, block_cols)
    idx, _ = pl.pallas_call(
        functools.partial(_sample_kernel, ncols=ncols, bc=block_cols,
                          sub=sub_cols),
        grid=(nblk,),
        in_specs=[pl.BlockSpec((rows, block_cols), lambda j: (0, j))],
        out_specs=[
            pl.BlockSpec((rows, 1), lambda j: (0, 0)),
            pl.BlockSpec((rows, 1), lambda j: (0, 0)),
        ],
        out_shape=[
            jax.ShapeDtypeStruct((rows, 1), jnp.int32),
            jax.ShapeDtypeStruct((rows, 1), jnp.float32),
        ],
    )(outputs)
    return idx


def kernel(outputs):
    return _sample(outputs)


# speedup vs baseline: 1.2563x; 1.2563x over previous
import functools

import numpy as np
import jax
import jax.numpy as jnp
from jax import lax
from jax.experimental import pallas as pl


def _i32(v) -> int:
    return int(np.uint32(v).view(np.int32))


_K1 = 0
_K2 = 42
_KS2 = _i32(np.uint32(_K1) ^ np.uint32(_K2) ^ np.uint32(0x1BD11BDA))
_ROTS = ((13, 15, 26, 6), (17, 29, 16, 24))
_KS = (_i32(_K1), _i32(_K2), _KS2)
_EXP_ONE = _i32(0x3F800000)
_BIG_I32 = np.iinfo(np.int32).max
_SUB = 1024


def _rotl(x, r):
    return lax.shift_left(x, np.int32(r)) | lax.shift_right_logical(
        x, np.int32(32 - r))


def _threefry_bits(i):
    x0 = jnp.full(i.shape, _KS[0], jnp.int32)
    x1 = i + np.int32(_KS[1])
    for rnd in range(5):
        for r in _ROTS[rnd % 2]:
            x0 = x0 + x1
            x1 = _rotl(x1, r)
            x1 = x1 ^ x0
        x0 = x0 + np.int32(_KS[(rnd + 1) % 3])
        x1 = x1 + np.int32(_i32(np.uint32(_KS[(rnd + 2) % 3]) + np.uint32(rnd + 1)))
    return x0 ^ x1


def _gumbel(i):
    bits = _threefry_bits(i)
    fb = lax.shift_right_logical(bits, np.int32(9)) | np.int32(_EXP_ONE)
    u = lax.bitcast_convert_type(fb, jnp.float32) - np.float32(1.0)
    return -jnp.log(-jnp.log(u))


def _subtile_scan(x_ref, ncols, col0_fn, mask_rem=None):
    rows, bc = x_ref.shape
    sub = _SUB if bc >= _SUB else bc
    bm_acc = bi_acc = None
    for s in range(bc // sub):
        x = x_ref[:, s * sub:(s + 1) * sub]
        lane = lax.broadcasted_iota(jnp.int32, x.shape, 1)
        row = lax.broadcasted_iota(jnp.int32, x.shape, 0)
        col0 = col0_fn(s)
        i = row * np.int32(ncols) + lane + col0
        v = x + _gumbel(i)
        if mask_rem is not None:
            v = jnp.where(lane + col0 < mask_rem, v, -jnp.inf)
        bm = jnp.max(v, axis=1, keepdims=True)
        cand = jnp.where(v == bm, lane, np.int32(_BIG_I32))
        bi = jnp.min(cand, axis=1, keepdims=True) + col0
        if bm_acc is None:
            bm_acc, bi_acc = bm, bi
        else:
            better = bm > bm_acc
            bm_acc = jnp.where(better, bm, bm_acc)
            bi_acc = jnp.where(better, bi, bi_acc)
    return bm_acc, bi_acc


def _update_running(j, bm, bi, idx_ref, val_ref):
    @pl.when(j == 0)
    def _():
        val_ref[...] = bm
        idx_ref[...] = bi

    @pl.when(j != 0)
    def _():
        better = bm > val_ref[...]
        val_ref[...] = jnp.where(better, bm, val_ref[...])
        idx_ref[...] = jnp.where(better, bi, idx_ref[...])


def _main_kernel(x_ref, idx_ref, val_ref, *, ncols, bc):
    j = pl.program_id(0)
    base_col = j * np.int32(bc)
    bm, bi = _subtile_scan(x_ref, ncols,
                           lambda s: base_col + np.int32(s * _SUB))
    _update_running(j, bm, bi, idx_ref, val_ref)


def _tail_kernel(x_ref, idx_ref, val_ref, *, ncols, col_lo, bc):
    j = pl.program_id(0)
    base_col = np.int32(col_lo) + j * np.int32(bc)
    bm, bi = _subtile_scan(x_ref, ncols,
                           lambda s: base_col + np.int32(s * _SUB),
                           mask_rem=np.int32(ncols))
    _update_running(j, bm, bi, idx_ref, val_ref)


@functools.partial(jax.jit, static_argnames=("block_cols",))
def _sample(outputs, block_cols=24576):
    rows, ncols = outputs.shape
    out_sds = [
        jax.ShapeDtypeStruct((rows, 1), jnp.int32),
        jax.ShapeDtypeStruct((rows, 1), jnp.float32),
    ]
    out_specs = [
        pl.BlockSpec((rows, 1), lambda j: (0, 0)),
        pl.BlockSpec((rows, 1), lambda j: (0, 0)),
    ]
    nmain = ncols // block_cols
    main_cols = nmain * block_cols
    if main_cols:
        idx_m, val_m = pl.pallas_call(
            functools.partial(_main_kernel, ncols=ncols, bc=block_cols),
            grid=(nmain,),
            in_specs=[pl.BlockSpec((rows, block_cols), lambda j: (0, j))],
            out_specs=out_specs,
            out_shape=out_sds,
        )(outputs)
        if main_cols == ncols:
            return idx_m
    else:
        idx_m = val_m = None

    tail_bc = _SUB
    assert main_cols % tail_bc == 0, (main_cols, tail_bc)
    tail_blk0 = main_cols // tail_bc
    ntail = pl.cdiv(ncols - main_cols, tail_bc)
    idx_t, val_t = pl.pallas_call(
        functools.partial(_tail_kernel, ncols=ncols, col_lo=main_cols,
                          bc=tail_bc),
        grid=(ntail,),
        in_specs=[pl.BlockSpec((rows, tail_bc),
                               lambda j: (0, tail_blk0 + j))],
        out_specs=out_specs,
        out_shape=out_sds,
    )(outputs)
    if idx_m is None:
        return idx_t
    better = val_t > val_m
    return jnp.where(better, idx_t, idx_m)


def kernel(outputs):
    return _sample(outputs)
